# Initial kernel scaffold; baseline (speedup 1.0000x reference)
import jax
import jax.numpy as jnp
from jax.experimental import pallas as pl


def kernel(x, keys, values):
    raise NotImplementedError("write your pallas kernel here")


# speedup vs baseline: 1.0901x; 1.0901x over previous
import functools

import jax
import jax.numpy as jnp
from jax import lax
from jax.experimental import pallas as pl
from jax.experimental.pallas import tpu as pltpu
from jax.experimental.pallas import tpu_sc as plsc

_ROW_BLOCK = 1024


def _argmin_body(xsq_ref, x_ref, keys_ref, ksq_ref, idx_ref):
    x = x_ref[...]
    keys = keys_ref[...]
    dot = lax.dot_general(
        x, keys, (((1,), (1,)), ((), ())),
        preferred_element_type=jnp.float32)
    d2 = xsq_ref[...] - 2.0 * dot + ksq_ref[...]
    dist = jnp.sqrt(jnp.maximum(d2, 0.0))
    idx_ref[...] = jnp.argmin(dist, axis=1).astype(jnp.int32)


def _nearest_code(flat, xsq, keys, ksq):
    n = flat.shape[0]
    k = keys.shape[0]
    grid = n // _ROW_BLOCK
    return pl.pallas_call(
        _argmin_body,
        grid=(grid,),
        in_specs=[
            pl.BlockSpec((_ROW_BLOCK, 1), lambda i: (i, 0)),
            pl.BlockSpec((_ROW_BLOCK, flat.shape[1]), lambda i: (i, 0)),
            pl.BlockSpec((k, keys.shape[1]), lambda i: (0, 0)),
            pl.BlockSpec((1, k), lambda i: (0, 0)),
        ],
        out_specs=pl.BlockSpec((_ROW_BLOCK,), lambda i: (i,)),
        out_shape=jax.ShapeDtypeStruct((n,), jnp.int32),
    )(xsq, flat, keys, ksq)


def _gather_rows(values, idx):
    n = idx.shape[0]
    out_c = values.shape[1]
    mesh = plsc.VectorSubcoreMesh(core_axis_name="c", subcore_axis_name="s")
    num_workers = 2 * 16
    b_per_w = n // num_workers

    @functools.partial(
        pl.kernel,
        mesh=mesh,
        out_type=jax.ShapeDtypeStruct((n, out_c), jnp.float32),
        scratch_types=[
            pltpu.VMEM((b_per_w,), jnp.int32),
            pltpu.VMEM((b_per_w, out_c), jnp.float32),
            pltpu.SemaphoreType.DMA,
        ],
    )
    def gather_kernel(values_hbm, idx_hbm, out_hbm, idx_v, rows_v, sem):
        wid = lax.axis_index("s") * 2 + lax.axis_index("c")
        base = wid * b_per_w
        pltpu.sync_copy(idx_hbm.at[pl.ds(base, b_per_w)], idx_v)
        pltpu.async_copy(values_hbm.at[idx_v], rows_v, sem).wait()
        pltpu.sync_copy(rows_v, out_hbm.at[pl.ds(base, b_per_w)])

    return gather_kernel(values, idx)


@jax.jit
def kernel(x, keys, values):
    batchsz, lenseq, in_c = x.shape
    flat = x.reshape(batchsz * lenseq, in_c)
    xsq = jnp.sum(flat * flat, axis=1, keepdims=True)
    ksq = jnp.sum(keys * keys, axis=1)[None, :]
    idx = _nearest_code(flat, xsq, keys, ksq)
    y = _gather_rows(values, idx)
    return y.reshape(batchsz, lenseq, values.shape[-1])


# speedup vs baseline: 1.1036x; 1.0124x over previous
import functools

import jax
import jax.numpy as jnp
from jax import lax
from jax.experimental import pallas as pl
from jax.experimental.pallas import tpu as pltpu
from jax.experimental.pallas import tpu_sc as plsc

_ROW_BLOCK = 1024


def _argmin_body(xsq_ref, x_ref, keysm2_ref, ksq_ref, idx_ref):
    x = x_ref[...]
    keysm2 = keysm2_ref[...]
    dot = lax.dot_general(
        x, keysm2, (((1,), (1,)), ((), ())),
        preferred_element_type=jnp.float32)
    d2 = xsq_ref[...] + dot + ksq_ref[...]
    dist = jnp.sqrt(jnp.maximum(d2, 0.0))
    idx_ref[...] = jnp.argmin(dist, axis=1).astype(jnp.int32)


def _nearest_code(flat, xsq, keys, ksq):
    n = flat.shape[0]
    k = keys.shape[0]
    grid = n // _ROW_BLOCK
    return pl.pallas_call(
        _argmin_body,
        grid=(grid,),
        in_specs=[
            pl.BlockSpec((_ROW_BLOCK, 1), lambda i: (i, 0)),
            pl.BlockSpec((_ROW_BLOCK, flat.shape[1]), lambda i: (i, 0)),
            pl.BlockSpec((k, keys.shape[1]), lambda i: (0, 0)),
            pl.BlockSpec((1, k), lambda i: (0, 0)),
        ],
        out_specs=pl.BlockSpec((_ROW_BLOCK,), lambda i: (i,)),
        out_shape=jax.ShapeDtypeStruct((n,), jnp.int32),
    )(xsq, flat, keys, ksq)


def _gather_rows(values, idx):
    n = idx.shape[0]
    out_c = values.shape[1]
    mesh = plsc.VectorSubcoreMesh(core_axis_name="c", subcore_axis_name="s")
    num_workers = 2 * 16
    b_per_w = n // num_workers

    @functools.partial(
        pl.kernel,
        mesh=mesh,
        out_type=jax.ShapeDtypeStruct((n, out_c), jnp.float32),
        scratch_types=[
            pltpu.VMEM((b_per_w,), jnp.int32),
            pltpu.VMEM((b_per_w, out_c), jnp.float32),
            pltpu.SemaphoreType.DMA,
        ],
    )
    def gather_kernel(values_hbm, idx_hbm, out_hbm, idx_v, rows_v, sem):
        wid = lax.axis_index("s") * 2 + lax.axis_index("c")
        base = wid * b_per_w
        pltpu.sync_copy(idx_hbm.at[pl.ds(base, b_per_w)], idx_v)
        pltpu.async_copy(values_hbm.at[idx_v], rows_v, sem).wait()
        pltpu.sync_copy(rows_v, out_hbm.at[pl.ds(base, b_per_w)])

    return gather_kernel(values, idx)


@jax.jit
def kernel(x, keys, values):
    batchsz, lenseq, in_c = x.shape
    flat = x.reshape(batchsz * lenseq, in_c)
    xsq = jnp.sum(flat * flat, axis=1, keepdims=True)
    ksq = jnp.sum(keys * keys, axis=1)[None, :]
    idx = _nearest_code(flat, xsq, keys * (-2.0), ksq)
    y = _gather_rows(values, idx)
    return y.reshape(batchsz, lenseq, values.shape[-1])


# speedup vs baseline: 1.2031x; 1.0902x over previous
import functools

import jax
import jax.numpy as jnp
from jax import lax
from jax.experimental import pallas as pl
from jax.experimental.pallas import tpu as pltpu
from jax.experimental.pallas import tpu_sc as plsc

_ROW_BLOCK = 1024


def _argmin_body(xsq_ref, x_ref, keysm2_ref, ksq_ref, idx_ref):
    x = x_ref[...]
    keysm2 = keysm2_ref[...]
    dot = lax.dot_general(
        x, keysm2, (((1,), (1,)), ((), ())),
        preferred_element_type=jnp.float32)
    xsq = xsq_ref[...].reshape(_ROW_BLOCK, 1)
    d2 = xsq + dot + ksq_ref[...]
    dist = jnp.sqrt(jnp.maximum(d2, 0.0))
    idx_ref[...] = jnp.argmin(dist, axis=1).astype(jnp.int32)


def _nearest_code(flat, xsq, keys, ksq):
    n = flat.shape[0]
    k = keys.shape[0]
    grid = n // _ROW_BLOCK
    return pl.pallas_call(
        _argmin_body,
        grid=(grid,),
        in_specs=[
            pl.BlockSpec((_ROW_BLOCK,), lambda i: (i,)),
            pl.BlockSpec((_ROW_BLOCK, flat.shape[1]), lambda i: (i, 0)),
            pl.BlockSpec((k, keys.shape[1]), lambda i: (0, 0)),
            pl.BlockSpec((1, k), lambda i: (0, 0)),
        ],
        out_specs=pl.BlockSpec((_ROW_BLOCK,), lambda i: (i,)),
        out_shape=jax.ShapeDtypeStruct((n,), jnp.int32),
    )(xsq, flat, keys, ksq)


def _gather_rows(values, idx):
    n = idx.shape[0]
    out_c = values.shape[1]
    mesh = plsc.VectorSubcoreMesh(core_axis_name="c", subcore_axis_name="s")
    num_workers = 2 * 16
    b_per_w = n // num_workers

    @functools.partial(
        pl.kernel,
        mesh=mesh,
        out_type=jax.ShapeDtypeStruct((n, out_c), jnp.float32),
        scratch_types=[
            pltpu.VMEM((b_per_w,), jnp.int32),
            pltpu.VMEM((b_per_w, out_c), jnp.float32),
            pltpu.SemaphoreType.DMA,
        ],
    )
    def gather_kernel(values_hbm, idx_hbm, out_hbm, idx_v, rows_v, sem):
        wid = lax.axis_index("s") * 2 + lax.axis_index("c")
        base = wid * b_per_w
        pltpu.sync_copy(idx_hbm.at[pl.ds(base, b_per_w)], idx_v)
        pltpu.async_copy(values_hbm.at[idx_v], rows_v, sem).wait()
        pltpu.sync_copy(rows_v, out_hbm.at[pl.ds(base, b_per_w)])

    return gather_kernel(values, idx)


@jax.jit
def kernel(x, keys, values):
    batchsz, lenseq, in_c = x.shape
    flat = lax.optimization_barrier(x.reshape(batchsz * lenseq, in_c))
    xsq = jnp.sum(flat * flat, axis=1)
    ksq = jnp.sum(keys * keys, axis=1)[None, :]
    idx = _nearest_code(flat, xsq, keys * (-2.0), ksq)
    y = _gather_rows(values, idx)
    return y.reshape(batchsz, lenseq, values.shape[-1])
